# Initial kernel scaffold; baseline (speedup 1.0000x reference)
import jax
import jax.numpy as jnp
from jax.experimental import pallas as pl


def kernel(x, support, mask, W_enc, b_enc, W_z0, b_z0, W_z1, b_z1, W_r0, b_r0, W_r1, b_r1, W_h0, b_h0, W_h1, b_h1):
    raise NotImplementedError("write your pallas kernel here")


# speedup vs baseline: 1.7124x; 1.7124x over previous
import jax
import jax.numpy as jnp
from jax.experimental import pallas as pl

_B, _N, _D = 32, 512, 128
_STEPS = 2


def _graph_gru_kernel(x_ref, sup_ref, mask_ref, w_enc_ref, b_enc_ref,
                      w_a_ref, b_a_ref, w_o_ref, b_o_ref, w_h1_ref, b_h1_ref,
                      out_ref):
    D = _D
    x = x_ref[0]
    sup = sup_ref[0]
    mask = mask_ref[0]

    h = jnp.dot(x, w_enc_ref[...], preferred_element_type=jnp.float32)
    out = mask * jnp.maximum(h + b_enc_ref[...], 0.0)

    for _ in range(_STEPS):
        a = jnp.dot(sup, out, preferred_element_type=jnp.float32)
        ga = jnp.dot(a, w_a_ref[...], preferred_element_type=jnp.float32) + b_a_ref[...]
        go = jnp.dot(out, w_o_ref[...], preferred_element_type=jnp.float32) + b_o_ref[...]
        z = jax.nn.sigmoid(ga[:, :D] + go[:, :D])
        r = jax.nn.sigmoid(ga[:, D:2 * D] + go[:, D:])
        h1 = jnp.dot(r * out, w_h1_ref[...], preferred_element_type=jnp.float32)
        hh = jnp.maximum(mask * (ga[:, 2 * D:] + h1 + b_h1_ref[...]), 0.0)
        out = hh * z + out * (1.0 - z)

    out_ref[0] = out


def kernel(x, support, mask, W_enc, b_enc, W_z0, b_z0, W_z1, b_z1,
           W_r0, b_r0, W_r1, b_r1, W_h0, b_h0, W_h1, b_h1):
    B, N, D = _B, _N, _D

    W_a = jnp.concatenate([W_z0, W_r0, W_h0], axis=1)
    b_a = jnp.concatenate([b_z0, b_r0, b_h0]).reshape(1, 3 * D)
    W_o = jnp.concatenate([W_z1, W_r1], axis=1)
    b_o = jnp.concatenate([b_z1, b_r1]).reshape(1, 2 * D)
    b_enc2 = b_enc.reshape(1, D)
    b_h1_2 = b_h1.reshape(1, D)

    batch_spec = lambda shape: pl.BlockSpec((1,) + shape, lambda b: (b,) + (0,) * len(shape))
    const_spec = lambda shape: pl.BlockSpec(shape, lambda b: (0,) * len(shape))

    return pl.pallas_call(
        _graph_gru_kernel,
        grid=(B,),
        in_specs=[
            batch_spec((N, D)),
            batch_spec((N, N)),
            batch_spec((N, 1)),
            const_spec((D, D)),
            const_spec((1, D)),
            const_spec((D, 3 * D)),
            const_spec((1, 3 * D)),
            const_spec((D, 2 * D)),
            const_spec((1, 2 * D)),
            const_spec((D, D)),
            const_spec((1, D)),
        ],
        out_specs=batch_spec((N, D)),
        out_shape=jax.ShapeDtypeStruct((B, N, D), jnp.float32),
    )(x, support, mask, W_enc, b_enc2, W_a, b_a, W_o, b_o, W_h1, b_h1_2)
